# baseline (speedup 1.0000x reference)
import functools

import jax
import jax.numpy as jnp
from jax import lax
from jax.experimental import pallas as pl
from jax.experimental.pallas import tpu as pltpu
from jax.experimental.pallas import tpu_sc as plsc

NC = 2
NS = 16
NW = NC * NS
CHUNK = 128


@functools.partial(jax.jit, static_argnames=("n_chunks", "dim"))
def _embedding_gather(x_resh, embedding, *, n_chunks, dim):
    total = NW * n_chunks * CHUNK
    mesh = plsc.VectorSubcoreMesh(core_axis_name="c", subcore_axis_name="s")

    @functools.partial(
        pl.kernel,
        out_type=jax.ShapeDtypeStruct((total, dim), jnp.float32),
        mesh=mesh,
        scratch_types=[
            pltpu.VMEM((n_chunks, CHUNK), jnp.int32),
            pltpu.VMEM((CHUNK, dim), jnp.float32),
            pltpu.VMEM((CHUNK, dim), jnp.float32),
            pltpu.SemaphoreType.DMA,
            pltpu.SemaphoreType.DMA,
        ],
        compiler_params=pltpu.CompilerParams(use_tc_tiling_on_sc=False),
    )
    def k(table_hbm, idx_hbm, out_hbm, idx_v, buf0, buf1, sem0, sem1):
        wid = lax.axis_index("s") * NC + lax.axis_index("c")
        base = wid * (n_chunks * CHUNK)
        pltpu.sync_copy(idx_hbm.at[wid], idx_v)

        bufs = (buf0, buf1)
        sems = (sem0, sem1)

        def start_gather(g, b):
            pltpu.async_copy(table_hbm.at[idx_v.at[g]], bufs[b], sems[b])

        def wait_gather(b):
            pltpu.make_async_copy(table_hbm.at[idx_v.at[0]], bufs[b], sems[b]).wait()

        def write_out(g, b):
            pltpu.sync_copy(bufs[b], out_hbm.at[pl.ds(base + g * CHUNK, CHUNK)])

        start_gather(0, 0)
        start_gather(1, 1)

        def body(g2, _):
            for b in range(2):
                g = g2 + b
                wait_gather(b)
                write_out(g, b)
                start_gather(g + 2, b)
            return 0

        lax.fori_loop(0, (n_chunks - 2) // 2, lambda i, c: body(2 * i, c), 0)

        for b in range(2):
            g = n_chunks - 2 + b
            wait_gather(b)
            write_out(g, b)

    return k(embedding, x_resh)


def kernel(x, embedding):
    b, l = x.shape
    dim = embedding.shape[1]
    n = b * l
    assert n % (NW * CHUNK) == 0
    n_chunks = n // (NW * CHUNK)
    x_resh = x.astype(jnp.int32).reshape(NW, n_chunks, CHUNK)
    out = _embedding_gather(x_resh, embedding, n_chunks=n_chunks, dim=dim)
    return out.reshape(b, l, dim)
